# baseline (speedup 1.0000x reference)
import functools

import numpy as np
import jax
import jax.numpy as jnp
from jax import lax
from jax.experimental import pallas as pl
from jax.experimental.pallas import tpu as pltpu
from jax.experimental.pallas import tpu_sc as plsc

_MW = 8192
_CK = 128
_RG = 32
_UNROLL = 4
_NLN2 = float(np.float32(-0.6931471805599453))


def _rotl(x, d):
    return (x << jnp.uint32(d)) | lax.shift_right_logical(x, jnp.uint32(32 - d))


def _threefry_gumbel(flat_i32):
    ks0 = jnp.uint32(0)
    ks1 = jnp.uint32(42)
    ks2 = jnp.uint32(0x1BD11BDA) ^ ks0 ^ ks1
    ks = (ks0, ks1, ks2)
    rots = ((13, 15, 26, 6), (17, 29, 16, 24))
    lo = flat_i32.astype(jnp.uint32)
    x1 = lo + ks1
    x0 = x1
    x1 = _rotl(x1, 13) ^ x0
    first = True
    for i in range(5):
        for r in rots[i % 2][1 if first else 0:]:
            x0 = x0 + x1
            x1 = _rotl(x1, r)
            x1 = x1 ^ x0
        first = False
        x0 = x0 + ks[(i + 1) % 3]
        x1 = x1 + ks[(i + 2) % 3] + jnp.uint32(i + 1)
    bits = x0 ^ x1
    fb = lax.bitcast_convert_type(
        (bits >> 9) | jnp.uint32(0x3F800000), jnp.float32) - jnp.float32(1.0)
    tiny = jnp.float32(np.finfo(np.float32).tiny)
    return -jnp.log(-jnp.log(fb + tiny))


def _run_groups(lp_ref, rv_s, rc_s, j, base, trips, tail, N, V):
    for g0 in range(0, N, _RG):
        row = g0 + lax.broadcasted_iota(jnp.int32, (_RG, _CK), 0)
        lane = lax.broadcasted_iota(jnp.int32, (_RG, _CK), 1)
        flat_base = row * V + base + lane

        def it(k, carry, flat_base=flat_base, g0=g0):
            rv, rc = carry
            off = k * _CK
            blk = lp_ref[pl.ds(g0, _RG), pl.ds(off, _CK)]
            flat = flat_base + off
            gum = _threefry_gumbel(flat)
            val = blk + gum
            upd = val > rv
            rv = jnp.where(upd, val, rv)
            rc = jnp.where(upd, flat, rc)
            return rv, rc

        rv0 = jnp.full((_RG, _CK), -jnp.inf, jnp.float32)
        rc0 = jnp.zeros((_RG, _CK), jnp.int32)
        rv, rc = lax.fori_loop(0, trips, it, (rv0, rc0), unroll=_UNROLL)

        if tail is not None:
            toff, nvalid = tail
            blk = lp_ref[pl.ds(g0, _RG), pl.ds(toff, _CK)]
            flat = flat_base + toff
            gum = _threefry_gumbel(flat)
            val = jnp.where(lane < nvalid, blk + gum, -jnp.inf)
            upd = val > rv
            rv = jnp.where(upd, val, rv)
            rc = jnp.where(upd, flat, rc)

        @pl.when(j == 0)
        def _(rv=rv, rc=rc, g0=g0):
            rv_s[pl.ds(g0, _RG), :] = rv
            rc_s[pl.ds(g0, _RG), :] = rc

        @pl.when(j > 0)
        def _(rv=rv, rc=rc, g0=g0):
            pv = rv_s[pl.ds(g0, _RG), :]
            pc = rc_s[pl.ds(g0, _RG), :]
            upd = rv > pv
            rv_s[pl.ds(g0, _RG), :] = jnp.where(upd, rv, pv)
            rc_s[pl.ds(g0, _RG), :] = jnp.where(upd, rc, pc)


def _body(lp_ref, lpprev_ref, yt_ref, lpn_ref, rv_s, rc_s, *, N, V, G):
    j = pl.program_id(0)
    last_w = V - (G - 1) * _MW
    last_full = last_w // _CK
    last_tail = last_w - last_full * _CK

    @pl.when(j < G - 1)
    def _():
        _run_groups(lp_ref, rv_s, rc_s, j, j * _MW, _MW // _CK, None, N, V)

    @pl.when(j == G - 1)
    def _():
        tail = (last_full * _CK, last_tail) if last_tail else None
        _run_groups(lp_ref, rv_s, rc_s, j, (G - 1) * _MW, last_full, tail,
                    N, V)

    @pl.when(j == G - 1)
    def _():
        rv = rv_s[...]
        rc = rc_s[...]
        m = jnp.max(rv, axis=1, keepdims=True)
        ties = rv == m
        big = jnp.int32(np.iinfo(np.int32).max)
        idx_flat = jnp.min(jnp.where(ties, rc, big), axis=1, keepdims=True)
        rowc = lax.broadcasted_iota(jnp.int32, (N, 1), 0)
        g_win = _threefry_gumbel(idx_flat)
        yt_ref[...] = idx_flat - rowc * V
        lpn_ref[...] = lpprev_ref[...] + (m - g_win)


_SC_WORKERS = 32


def _sc_scatter_body(yprev_hbm, lens_hbm, yt_hbm, out_hbm,
                     buf, lens_v, yt_v, *, S, N, rows_per):
    wid = lax.axis_index("s") * 2 + lax.axis_index("c")
    r0 = wid * rows_per
    pltpu.sync_copy(yprev_hbm.at[pl.ds(r0, rows_per)], buf)
    pltpu.sync_copy(lens_hbm, lens_v)
    pltpu.sync_copy(yt_hbm, yt_v)

    def row_fn(r, _):
        def chunk_fn(c8, _):
            c0 = c8 * 16
            lc = lens_v[pl.ds(c0, 16)]
            cond = lc == r0 + r
            yc = yt_v[pl.ds(c0, 16)]
            x = buf[r, pl.ds(c0, 16)]
            buf[r, pl.ds(c0, 16)] = jnp.where(cond, yc, x)
            return 0
        return lax.fori_loop(0, N // 16, chunk_fn, 0)

    lax.fori_loop(0, rows_per, row_fn, 0)
    pltpu.sync_copy(buf, out_hbm.at[pl.ds(r0, rows_per)])


def _sc_scatter(y_prev, y_prev_lens, yt):
    S, N = y_prev.shape
    rows_per = S // _SC_WORKERS
    mesh = plsc.VectorSubcoreMesh(core_axis_name="c", subcore_axis_name="s")
    return pl.kernel(
        functools.partial(_sc_scatter_body, S=S, N=N, rows_per=rows_per),
        mesh=mesh,
        out_type=jax.ShapeDtypeStruct((S, N), y_prev.dtype),
        scratch_types=[
            pltpu.VMEM((rows_per, N), jnp.int32),
            pltpu.VMEM((N,), jnp.int32),
            pltpu.VMEM((N,), jnp.int32),
        ],
    )(y_prev, y_prev_lens, yt)


def kernel(log_probs_t, log_probs_prev, y_prev, y_prev_lens):
    N, V = log_probs_t.shape
    S = y_prev.shape[0]
    G = pl.cdiv(V, _MW)
    yt, lpn = pl.pallas_call(
        functools.partial(_body, N=N, V=V, G=G),
        grid=(G,),
        in_specs=[
            pl.BlockSpec((N, _MW), lambda j: (0, j)),
            pl.BlockSpec((N, 1), lambda j: (0, 0)),
        ],
        out_specs=[
            pl.BlockSpec((N, 1), lambda j: (0, 0)),
            pl.BlockSpec((N, 1), lambda j: (0, 0)),
        ],
        out_shape=[
            jax.ShapeDtypeStruct((N, 1), jnp.int32),
            jax.ShapeDtypeStruct((N, 1), jnp.float32),
        ],
        scratch_shapes=[
            pltpu.VMEM((N, _CK), jnp.float32),
            pltpu.VMEM((N, _CK), jnp.int32),
        ],
    )(log_probs_t, log_probs_prev.reshape(N, 1))

    y_next = _sc_scatter(y_prev, y_prev_lens,
                         yt.reshape(N).astype(y_prev.dtype))
    return y_next, lpn.reshape(N)


# speedup vs baseline: 1.0762x; 1.0762x over previous
import functools

import numpy as np
import jax
import jax.numpy as jnp
from jax import lax
from jax.experimental import pallas as pl
from jax.experimental.pallas import tpu as pltpu

_MW = 8192
_CK = 128
_RG = 32
_UNROLL = 4


def _rotl(x, d):
    return (x << jnp.uint32(d)) | lax.shift_right_logical(x, jnp.uint32(32 - d))


def _threefry_gumbel(flat_i32):
    ks0 = jnp.uint32(0)
    ks1 = jnp.uint32(42)
    ks2 = jnp.uint32(0x1BD11BDA) ^ ks0 ^ ks1
    ks = (ks0, ks1, ks2)
    rots = ((13, 15, 26, 6), (17, 29, 16, 24))
    lo = flat_i32.astype(jnp.uint32)
    x1 = lo + ks1
    x0 = x1
    x1 = _rotl(x1, 13) ^ x0
    first = True
    for i in range(5):
        for r in rots[i % 2][1 if first else 0:]:
            x0 = x0 + x1
            x1 = _rotl(x1, r)
            x1 = x1 ^ x0
        first = False
        x0 = x0 + ks[(i + 1) % 3]
        x1 = x1 + ks[(i + 2) % 3] + jnp.uint32(i + 1)
    bits = x0 ^ x1
    fb = lax.bitcast_convert_type(
        (bits >> 9) | jnp.uint32(0x3F800000), jnp.float32) - jnp.float32(1.0)
    tiny = jnp.float32(np.finfo(np.float32).tiny)
    return -jnp.log(-jnp.log(fb + tiny))


def _run_groups(lp_ref, rv_s, rc_s, j, base, trips, tail, N, V):
    for g0 in range(0, N, _RG):
        row = g0 + lax.broadcasted_iota(jnp.int32, (_RG, _CK), 0)
        lane = lax.broadcasted_iota(jnp.int32, (_RG, _CK), 1)
        flat_base = row * V + base + lane

        def it(k, carry, flat_base=flat_base, g0=g0):
            rv, rc = carry
            off = k * _CK
            blk = lp_ref[pl.ds(g0, _RG), pl.ds(off, _CK)]
            flat = flat_base + off
            gum = _threefry_gumbel(flat)
            val = blk + gum
            upd = val > rv
            rv = jnp.where(upd, val, rv)
            rc = jnp.where(upd, flat, rc)
            return rv, rc

        rv0 = jnp.full((_RG, _CK), -jnp.inf, jnp.float32)
        rc0 = jnp.zeros((_RG, _CK), jnp.int32)
        rv, rc = lax.fori_loop(0, trips, it, (rv0, rc0), unroll=_UNROLL)

        if tail is not None:
            toff, nvalid = tail
            blk = lp_ref[pl.ds(g0, _RG), pl.ds(toff, _CK)]
            flat = flat_base + toff
            gum = _threefry_gumbel(flat)
            val = jnp.where(lane < nvalid, blk + gum, -jnp.inf)
            upd = val > rv
            rv = jnp.where(upd, val, rv)
            rc = jnp.where(upd, flat, rc)

        @pl.when(j == 0)
        def _(rv=rv, rc=rc, g0=g0):
            rv_s[pl.ds(g0, _RG), :] = rv
            rc_s[pl.ds(g0, _RG), :] = rc

        @pl.when(j > 0)
        def _(rv=rv, rc=rc, g0=g0):
            pv = rv_s[pl.ds(g0, _RG), :]
            pc = rc_s[pl.ds(g0, _RG), :]
            upd = rv > pv
            rv_s[pl.ds(g0, _RG), :] = jnp.where(upd, rv, pv)
            rc_s[pl.ds(g0, _RG), :] = jnp.where(upd, rc, pc)


def _body(lp_ref, lpprev_ref, lens_ref, yprev_ref,
          lpn_ref, ynext_ref, rv_s, rc_s, *, N, V, G):
    j = pl.program_id(0)
    last_w = V - (G - 1) * _MW
    last_full = last_w // _CK
    last_tail = last_w - last_full * _CK

    @pl.when(j < G - 1)
    def _():
        _run_groups(lp_ref, rv_s, rc_s, j, j * _MW, _MW // _CK, None, N, V)

    @pl.when(j == G - 1)
    def _():
        tail = (last_full * _CK, last_tail) if last_tail else None
        _run_groups(lp_ref, rv_s, rc_s, j, (G - 1) * _MW, last_full, tail,
                    N, V)

    @pl.when(j == G - 1)
    def _():
        rv = rv_s[...]
        rc = rc_s[...]
        m = jnp.max(rv, axis=1, keepdims=True)
        ties = rv == m
        big = jnp.int32(np.iinfo(np.int32).max)
        idx_flat = jnp.min(jnp.where(ties, rc, big), axis=1, keepdims=True)
        rowc = lax.broadcasted_iota(jnp.int32, (N, 1), 0)
        g_win = _threefry_gumbel(idx_flat)
        idx = idx_flat - rowc * V
        lpn_ref[...] = lpprev_ref[...] + (m - g_win)
        S = ynext_ref.shape[0]
        yt_row = jnp.transpose(idx, (1, 0))
        rowi = lax.broadcasted_iota(jnp.int32, (S, N), 0)
        ynext_ref[...] = jnp.where(rowi == lens_ref[...], yt_row,
                                   yprev_ref[...])


def kernel(log_probs_t, log_probs_prev, y_prev, y_prev_lens):
    N, V = log_probs_t.shape
    S = y_prev.shape[0]
    G = pl.cdiv(V, _MW)
    lpn, y_next = pl.pallas_call(
        functools.partial(_body, N=N, V=V, G=G),
        grid=(G,),
        in_specs=[
            pl.BlockSpec((N, _MW), lambda j: (0, j)),
            pl.BlockSpec((N, 1), lambda j: (0, 0)),
            pl.BlockSpec((1, N), lambda j: (0, 0)),
            pl.BlockSpec((S, N), lambda j: (0, 0)),
        ],
        out_specs=[
            pl.BlockSpec((N, 1), lambda j: (0, 0)),
            pl.BlockSpec((S, N), lambda j: (0, 0)),
        ],
        out_shape=[
            jax.ShapeDtypeStruct((N, 1), jnp.float32),
            jax.ShapeDtypeStruct((S, N), y_prev.dtype),
        ],
        scratch_shapes=[
            pltpu.VMEM((N, _CK), jnp.float32),
            pltpu.VMEM((N, _CK), jnp.int32),
        ],
    )(log_probs_t, log_probs_prev.reshape(N, 1),
      y_prev_lens.reshape(1, N), y_prev)
    return y_next, lpn.reshape(N)


# speedup vs baseline: 1.0902x; 1.0130x over previous
import functools

import numpy as np
import jax
import jax.numpy as jnp
from jax import lax
from jax.experimental import pallas as pl
from jax.experimental.pallas import tpu as pltpu

_MW = 8192
_CK = 128
_RG = 64
_UNROLL = 2


def _rotl(x, d):
    return (x << jnp.uint32(d)) | lax.shift_right_logical(x, jnp.uint32(32 - d))


def _threefry_gumbel(flat_i32):
    ks0 = jnp.uint32(0)
    ks1 = jnp.uint32(42)
    ks2 = jnp.uint32(0x1BD11BDA) ^ ks0 ^ ks1
    ks = (ks0, ks1, ks2)
    rots = ((13, 15, 26, 6), (17, 29, 16, 24))
    lo = flat_i32.astype(jnp.uint32)
    x1 = lo + ks1
    x0 = x1
    x1 = _rotl(x1, 13) ^ x0
    first = True
    for i in range(5):
        for r in rots[i % 2][1 if first else 0:]:
            x0 = x0 + x1
            x1 = _rotl(x1, r)
            x1 = x1 ^ x0
        first = False
        x0 = x0 + ks[(i + 1) % 3]
        x1 = x1 + ks[(i + 2) % 3] + jnp.uint32(i + 1)
    bits = x0 ^ x1
    fb = lax.bitcast_convert_type(
        (bits >> 9) | jnp.uint32(0x3F800000), jnp.float32) - jnp.float32(1.0)
    tiny = jnp.float32(np.finfo(np.float32).tiny)
    return -jnp.log(-jnp.log(fb + tiny))


def _run_groups(lp_ref, rv_s, rc_s, j, base, trips, tail, N, V):
    for g0 in range(0, N, _RG):
        row = g0 + lax.broadcasted_iota(jnp.int32, (_RG, _CK), 0)
        lane = lax.broadcasted_iota(jnp.int32, (_RG, _CK), 1)
        flat_base = row * V + base + lane

        def it(k, carry, flat_base=flat_base, g0=g0):
            rv, rc = carry
            off = k * _CK
            blk = lp_ref[pl.ds(g0, _RG), pl.ds(off, _CK)]
            flat = flat_base + off
            gum = _threefry_gumbel(flat)
            val = blk + gum
            upd = val > rv
            rv = jnp.where(upd, val, rv)
            rc = jnp.where(upd, flat, rc)
            return rv, rc

        rv0 = jnp.full((_RG, _CK), -jnp.inf, jnp.float32)
        rc0 = jnp.zeros((_RG, _CK), jnp.int32)
        rv, rc = lax.fori_loop(0, trips, it, (rv0, rc0), unroll=_UNROLL)

        if tail is not None:
            toff, nvalid = tail
            blk = lp_ref[pl.ds(g0, _RG), pl.ds(toff, _CK)]
            flat = flat_base + toff
            gum = _threefry_gumbel(flat)
            val = jnp.where(lane < nvalid, blk + gum, -jnp.inf)
            upd = val > rv
            rv = jnp.where(upd, val, rv)
            rc = jnp.where(upd, flat, rc)

        @pl.when(j == 0)
        def _(rv=rv, rc=rc, g0=g0):
            rv_s[pl.ds(g0, _RG), :] = rv
            rc_s[pl.ds(g0, _RG), :] = rc

        @pl.when(j > 0)
        def _(rv=rv, rc=rc, g0=g0):
            pv = rv_s[pl.ds(g0, _RG), :]
            pc = rc_s[pl.ds(g0, _RG), :]
            upd = rv > pv
            rv_s[pl.ds(g0, _RG), :] = jnp.where(upd, rv, pv)
            rc_s[pl.ds(g0, _RG), :] = jnp.where(upd, rc, pc)


def _body(lp_ref, lpprev_ref, lens_ref, yprev_ref,
          lpn_ref, ynext_ref, rv_s, rc_s, *, N, V, G):
    j = pl.program_id(0)
    last_w = V - (G - 1) * _MW
    last_full = last_w // _CK
    last_tail = last_w - last_full * _CK

    @pl.when(j < G - 1)
    def _():
        _run_groups(lp_ref, rv_s, rc_s, j, j * _MW, _MW // _CK, None, N, V)

    @pl.when(j == G - 1)
    def _():
        tail = (last_full * _CK, last_tail) if last_tail else None
        _run_groups(lp_ref, rv_s, rc_s, j, (G - 1) * _MW, last_full, tail,
                    N, V)

    @pl.when(j == G - 1)
    def _():
        rv = rv_s[...]
        rc = rc_s[...]
        m = jnp.max(rv, axis=1, keepdims=True)
        ties = rv == m
        big = jnp.int32(np.iinfo(np.int32).max)
        idx_flat = jnp.min(jnp.where(ties, rc, big), axis=1, keepdims=True)
        rowc = lax.broadcasted_iota(jnp.int32, (N, 1), 0)
        g_win = _threefry_gumbel(idx_flat)
        idx = idx_flat - rowc * V
        lpn_ref[...] = lpprev_ref[...] + (m - g_win)
        S = ynext_ref.shape[0]
        yt_row = jnp.transpose(idx, (1, 0))
        rowi = lax.broadcasted_iota(jnp.int32, (S, N), 0)
        ynext_ref[...] = jnp.where(rowi == lens_ref[...], yt_row,
                                   yprev_ref[...])


def kernel(log_probs_t, log_probs_prev, y_prev, y_prev_lens):
    N, V = log_probs_t.shape
    S = y_prev.shape[0]
    G = pl.cdiv(V, _MW)
    lpn, y_next = pl.pallas_call(
        functools.partial(_body, N=N, V=V, G=G),
        grid=(G,),
        in_specs=[
            pl.BlockSpec((N, _MW), lambda j: (0, j)),
            pl.BlockSpec((N, 1), lambda j: (0, 0)),
            pl.BlockSpec((1, N), lambda j: (0, 0)),
            pl.BlockSpec((S, N), lambda j: (0, 0)),
        ],
        out_specs=[
            pl.BlockSpec((N, 1), lambda j: (0, 0)),
            pl.BlockSpec((S, N), lambda j: (0, 0)),
        ],
        out_shape=[
            jax.ShapeDtypeStruct((N, 1), jnp.float32),
            jax.ShapeDtypeStruct((S, N), y_prev.dtype),
        ],
        scratch_shapes=[
            pltpu.VMEM((N, _CK), jnp.float32),
            pltpu.VMEM((N, _CK), jnp.int32),
        ],
    )(log_probs_t, log_probs_prev.reshape(N, 1),
      y_prev_lens.reshape(1, N), y_prev)
    return y_next, lpn.reshape(N)
